# baseline (speedup 1.0000x reference)
import functools

import jax
import jax.numpy as jnp
from jax import lax
from jax.experimental import pallas as pl
from jax.experimental.pallas import tpu as pltpu
from jax.experimental.pallas import tpu_sc as plsc

D_IN = 768
D_LAT = 64
K = 128

T = 32 * 1024
BT = 2048
NB = T // BT

NB_SC = 4
T_SC = NB_SC * BT

NW = 32
TPW = T_SC // NW
CH = 64
NCH = TPW // CH


def _vq_head(x, encw, encb, cb):
    z = lax.dot_general(x, encw, (((1,), (1,)), ((), ())),
                        preferred_element_type=jnp.float32)
    z = z + encb
    dots = lax.dot_general(z, cb, (((1,), (1,)), ((), ())),
                           preferred_element_type=jnp.float32)
    z2 = jnp.sum(z * z, axis=-1, keepdims=True)
    e2 = jnp.sum(cb * cb, axis=-1)
    dist = z2 - 2.0 * dots + e2[None, :]
    mind = jnp.min(dist, axis=-1, keepdims=True)
    kiota = lax.broadcasted_iota(jnp.int32, dist.shape, 1)
    idx = jnp.min(jnp.where(dist == mind, kiota, K), axis=-1)
    onehot = (kiota == idx[:, None]).astype(jnp.float32)
    return idx, onehot, jnp.sum(mind)


def _tc_a_body(x_ref, encw_ref, encb_ref, cb_ref, decw_ref, decb_ref,
               idx_ref, loss_ref, deccb_ref):
    i = pl.program_id(0)
    idx, _, part = _vq_head(x_ref[...], encw_ref[...], encb_ref[...],
                            cb_ref[...])
    idx_ref[...] = idx

    @pl.when(i == 0)
    def _():
        loss_ref[...] = jnp.zeros((1, 1), jnp.float32)
        deccb = lax.dot_general(
            cb_ref[...], decw_ref[...], (((1,), (1,)), ((), ())),
            preferred_element_type=jnp.float32)
        deccb_ref[...] = deccb + decb_ref[...]

    loss_ref[...] += part[None, None]


def _tc_a(xf, enc_W, enc_b2, dec_W, dec_b2, codebook):
    return pl.pallas_call(
        _tc_a_body,
        grid=(NB_SC,),
        in_specs=[
            pl.BlockSpec((BT, D_IN), lambda i: (i, 0)),
            pl.BlockSpec((D_LAT, D_IN), lambda i: (0, 0)),
            pl.BlockSpec((1, D_LAT), lambda i: (0, 0)),
            pl.BlockSpec((K, D_LAT), lambda i: (0, 0)),
            pl.BlockSpec((D_IN, D_LAT), lambda i: (0, 0)),
            pl.BlockSpec((1, D_IN), lambda i: (0, 0)),
        ],
        out_specs=[
            pl.BlockSpec((BT,), lambda i: (i,)),
            pl.BlockSpec((1, 1), lambda i: (0, 0)),
            pl.BlockSpec((K, D_IN), lambda i: (0, 0)),
        ],
        out_shape=[
            jax.ShapeDtypeStruct((T_SC,), jnp.int32),
            jax.ShapeDtypeStruct((1, 1), jnp.float32),
            jax.ShapeDtypeStruct((K, D_IN), jnp.float32),
        ],
    )(xf, enc_W, enc_b2, codebook, dec_W, dec_b2)


def _tc_b_body(x_ref, encw_ref, encb_ref, cb_ref, deccb_ref,
               idx_ref, loss_ref, out_ref):
    i = pl.program_id(0)
    idx, onehot, part = _vq_head(x_ref[...], encw_ref[...], encb_ref[...],
                                 cb_ref[...])
    idx_ref[...] = idx
    out_ref[...] = lax.dot_general(
        onehot, deccb_ref[...], (((1,), (0,)), ((), ())),
        preferred_element_type=jnp.float32)

    @pl.when(i == 0)
    def _():
        loss_ref[...] = jnp.zeros((1, 1), jnp.float32)

    loss_ref[...] += part[None, None]


def _tc_b(xf, enc_W, enc_b2, codebook, dec_cb):
    return pl.pallas_call(
        _tc_b_body,
        grid=(NB - NB_SC,),
        in_specs=[
            pl.BlockSpec((BT, D_IN), lambda i: (i + NB_SC, 0)),
            pl.BlockSpec((D_LAT, D_IN), lambda i: (0, 0)),
            pl.BlockSpec((1, D_LAT), lambda i: (0, 0)),
            pl.BlockSpec((K, D_LAT), lambda i: (0, 0)),
            pl.BlockSpec((K, D_IN), lambda i: (0, 0)),
        ],
        out_specs=[
            pl.BlockSpec((BT,), lambda i: (i,)),
            pl.BlockSpec((1, 1), lambda i: (0, 0)),
            pl.BlockSpec((BT, D_IN), lambda i: (i + NB_SC, 0)),
        ],
        out_shape=[
            jax.ShapeDtypeStruct((T - T_SC,), jnp.int32),
            jax.ShapeDtypeStruct((1, 1), jnp.float32),
            jax.ShapeDtypeStruct((T, D_IN), jnp.float32),
        ],
    )(xf, enc_W, enc_b2, codebook, dec_cb)


def _sc_gather(dec_cb, idx3):
    mesh = plsc.VectorSubcoreMesh(core_axis_name="c", subcore_axis_name="s")

    @functools.partial(
        pl.kernel,
        out_type=jax.ShapeDtypeStruct((T_SC, D_IN), jnp.float32),
        mesh=mesh,
        scratch_types=(
            [pltpu.VMEM((NCH, CH), jnp.int32)]
            + [pltpu.VMEM((CH, D_IN), jnp.float32) for _ in range(2)]
            + [pltpu.SemaphoreType.DMA for _ in range(4)]
        ),
    )
    def body(deccb_hbm, idx_hbm, out_hbm, idx_v, buf0, buf1,
             gsem0, gsem1, psem0, psem1):
        wid = lax.axis_index("s") * 2 + lax.axis_index("c")
        pltpu.sync_copy(idx_hbm.at[wid], idx_v)
        base = wid * TPW
        bufs = (buf0, buf1)
        gsems = (gsem0, gsem1)
        psems = (psem0, psem1)
        gather = [None, None]
        put = [None, None]
        gather[0] = pltpu.async_copy(deccb_hbm.at[idx_v.at[0]], buf0, gsem0)
        for c in range(NCH):
            b = c & 1
            nb = 1 - b
            if c + 1 < NCH:
                if put[nb] is not None:
                    put[nb].wait()
                    put[nb] = None
                gather[nb] = pltpu.async_copy(
                    deccb_hbm.at[idx_v.at[c + 1]], bufs[nb], gsems[nb])
            gather[b].wait()
            put[b] = pltpu.async_copy(
                bufs[b], out_hbm.at[pl.ds(base + c * CH, CH)], psems[b])
        for b in (0, 1):
            if put[b] is not None:
                put[b].wait()

    return body(dec_cb, idx3)


def _merge_body(sc_ref, alias_ref, out_ref):
    out_ref[...] = sc_ref[...]


def _merge_tc(out_sc, out_b):
    return pl.pallas_call(
        _merge_body,
        grid=(NB_SC,),
        in_specs=[
            pl.BlockSpec((BT, D_IN), lambda i: (i, 0)),
            pl.BlockSpec(memory_space=pl.ANY),
        ],
        out_specs=pl.BlockSpec((BT, D_IN), lambda i: (i, 0)),
        out_shape=jax.ShapeDtypeStruct((T, D_IN), jnp.float32),
        input_output_aliases={1: 0},
    )(out_sc, out_b)


def kernel(x, enc_W, enc_b, dec_W, dec_b, codebook):
    B, N, _ = x.shape
    xf = x.reshape(T, D_IN)
    enc_b2 = enc_b.reshape(1, D_LAT)
    dec_b2 = dec_b.reshape(1, D_IN)
    idx_a, loss_a, dec_cb = _tc_a(xf, enc_W, enc_b2, dec_W, dec_b2, codebook)
    idx3 = idx_a.reshape(NW, NCH, CH)
    out_sc = _sc_gather(dec_cb, idx3)
    idx_b, loss_b, out_b = _tc_b(xf, enc_W, enc_b2, codebook, dec_cb)
    out_flat = _merge_tc(out_sc, out_b)
    out = out_flat.reshape(B, N, D_IN)
    indices = jnp.concatenate([idx_a, idx_b]).reshape(B, N)
    commit_loss = (loss_a[0, 0] + loss_b[0, 0]) / jnp.float32(T * D_LAT)
    return out, indices, commit_loss


# speedup vs baseline: 1.4665x; 1.4665x over previous
import functools

import jax
import jax.numpy as jnp
from jax import lax
from jax.experimental import pallas as pl
from jax.experimental.pallas import tpu as pltpu
from jax.experimental.pallas import tpu_sc as plsc

D_IN = 768
D_LAT = 64
K = 128

T = 32 * 1024
BT = 2048
NB = T // BT

NB_SC = 1
T_SC = NB_SC * BT

NW = 32
TPW = T_SC // NW
CH = 64
NCH = TPW // CH


def _vq_head(x, encw, encb, cb):
    z = lax.dot_general(x, encw, (((1,), (1,)), ((), ())),
                        preferred_element_type=jnp.float32)
    z = z + encb
    dots = lax.dot_general(z, cb, (((1,), (1,)), ((), ())),
                           preferred_element_type=jnp.float32)
    z2 = jnp.sum(z * z, axis=-1, keepdims=True)
    e2 = jnp.sum(cb * cb, axis=-1)
    dist = z2 - 2.0 * dots + e2[None, :]
    mind = jnp.min(dist, axis=-1, keepdims=True)
    kiota = lax.broadcasted_iota(jnp.int32, dist.shape, 1)
    idx = jnp.min(jnp.where(dist == mind, kiota, K), axis=-1)
    onehot = (kiota == idx[:, None]).astype(jnp.float32)
    return idx, onehot, jnp.sum(mind)


def _tc_a_body(x_ref, encw_ref, encb_ref, cb_ref, decw_ref, decb_ref,
               idx_ref, loss_ref, deccb_ref):
    i = pl.program_id(0)
    idx, _, part = _vq_head(x_ref[...], encw_ref[...], encb_ref[...],
                            cb_ref[...])
    idx_ref[...] = idx

    @pl.when(i == 0)
    def _():
        loss_ref[...] = jnp.zeros((1, 1), jnp.float32)
        deccb = lax.dot_general(
            cb_ref[...], decw_ref[...], (((1,), (1,)), ((), ())),
            preferred_element_type=jnp.float32)
        deccb_ref[...] = deccb + decb_ref[...]

    loss_ref[...] += part[None, None]


def _tc_a(xf, enc_W, enc_b2, dec_W, dec_b2, codebook):
    return pl.pallas_call(
        _tc_a_body,
        grid=(NB_SC,),
        in_specs=[
            pl.BlockSpec((BT, D_IN), lambda i: (i, 0)),
            pl.BlockSpec((D_LAT, D_IN), lambda i: (0, 0)),
            pl.BlockSpec((1, D_LAT), lambda i: (0, 0)),
            pl.BlockSpec((K, D_LAT), lambda i: (0, 0)),
            pl.BlockSpec((D_IN, D_LAT), lambda i: (0, 0)),
            pl.BlockSpec((1, D_IN), lambda i: (0, 0)),
        ],
        out_specs=[
            pl.BlockSpec((BT,), lambda i: (i,)),
            pl.BlockSpec((1, 1), lambda i: (0, 0)),
            pl.BlockSpec((K, D_IN), lambda i: (0, 0)),
        ],
        out_shape=[
            jax.ShapeDtypeStruct((T_SC,), jnp.int32),
            jax.ShapeDtypeStruct((1, 1), jnp.float32),
            jax.ShapeDtypeStruct((K, D_IN), jnp.float32),
        ],
    )(xf, enc_W, enc_b2, codebook, dec_W, dec_b2)


def _tc_b_body(x_ref, encw_ref, encb_ref, cb_ref, deccb_ref, alias_ref,
               idx_ref, loss_ref, out_ref):
    i = pl.program_id(0)
    idx, onehot, part = _vq_head(x_ref[...], encw_ref[...], encb_ref[...],
                                 cb_ref[...])
    idx_ref[...] = idx
    out_ref[...] = lax.dot_general(
        onehot, deccb_ref[...], (((1,), (0,)), ((), ())),
        preferred_element_type=jnp.float32)

    @pl.when(i == 0)
    def _():
        loss_ref[...] = jnp.zeros((1, 1), jnp.float32)

    loss_ref[...] += part[None, None]


def _tc_b(xf, enc_W, enc_b2, codebook, dec_cb, out_sc):
    return pl.pallas_call(
        _tc_b_body,
        grid=(NB - NB_SC,),
        in_specs=[
            pl.BlockSpec((BT, D_IN), lambda i: (i + NB_SC, 0)),
            pl.BlockSpec((D_LAT, D_IN), lambda i: (0, 0)),
            pl.BlockSpec((1, D_LAT), lambda i: (0, 0)),
            pl.BlockSpec((K, D_LAT), lambda i: (0, 0)),
            pl.BlockSpec((K, D_IN), lambda i: (0, 0)),
            pl.BlockSpec(memory_space=pl.ANY),
        ],
        out_specs=[
            pl.BlockSpec((BT,), lambda i: (i,)),
            pl.BlockSpec((1, 1), lambda i: (0, 0)),
            pl.BlockSpec((BT, D_IN), lambda i: (i + NB_SC, 0)),
        ],
        out_shape=[
            jax.ShapeDtypeStruct((T - T_SC,), jnp.int32),
            jax.ShapeDtypeStruct((1, 1), jnp.float32),
            jax.ShapeDtypeStruct((T, D_IN), jnp.float32),
        ],
        input_output_aliases={5: 2},
    )(xf, enc_W, enc_b2, codebook, dec_cb, out_sc)


def _sc_gather(dec_cb, idx3):
    mesh = plsc.VectorSubcoreMesh(core_axis_name="c", subcore_axis_name="s")

    @functools.partial(
        pl.kernel,
        out_type=jax.ShapeDtypeStruct((T, D_IN), jnp.float32),
        mesh=mesh,
        scratch_types=(
            [pltpu.VMEM((NCH, CH), jnp.int32)]
            + [pltpu.VMEM((CH, D_IN), jnp.float32) for _ in range(2)]
            + [pltpu.SemaphoreType.DMA for _ in range(4)]
        ),
    )
    def body(deccb_hbm, idx_hbm, out_hbm, idx_v, buf0, buf1,
             gsem0, gsem1, psem0, psem1):
        wid = lax.axis_index("s") * 2 + lax.axis_index("c")
        pltpu.sync_copy(idx_hbm.at[wid], idx_v)
        base = wid * TPW
        bufs = (buf0, buf1)
        gsems = (gsem0, gsem1)
        psems = (psem0, psem1)
        gather = [None, None]
        put = [None, None]
        gather[0] = pltpu.async_copy(deccb_hbm.at[idx_v.at[0]], buf0, gsem0)
        for c in range(NCH):
            b = c & 1
            nb = 1 - b
            if c + 1 < NCH:
                if put[nb] is not None:
                    put[nb].wait()
                    put[nb] = None
                gather[nb] = pltpu.async_copy(
                    deccb_hbm.at[idx_v.at[c + 1]], bufs[nb], gsems[nb])
            gather[b].wait()
            put[b] = pltpu.async_copy(
                bufs[b], out_hbm.at[pl.ds(base + c * CH, CH)], psems[b])
        for b in (0, 1):
            if put[b] is not None:
                put[b].wait()

    return body(dec_cb, idx3)


def kernel(x, enc_W, enc_b, dec_W, dec_b, codebook):
    B, N, _ = x.shape
    xf = x.reshape(T, D_IN)
    enc_b2 = enc_b.reshape(1, D_LAT)
    dec_b2 = dec_b.reshape(1, D_IN)
    idx_a, loss_a, dec_cb = _tc_a(xf, enc_W, enc_b2, dec_W, dec_b2, codebook)
    idx3 = idx_a.reshape(NW, NCH, CH)
    out_sc = _sc_gather(dec_cb, idx3)
    idx_b, loss_b, out_flat = _tc_b(xf, enc_W, enc_b2, codebook, dec_cb,
                                    out_sc)
    out = out_flat.reshape(B, N, D_IN)
    indices = jnp.concatenate([idx_a, idx_b]).reshape(B, N)
    commit_loss = (loss_a[0, 0] + loss_b[0, 0]) / jnp.float32(T * D_LAT)
    return out, indices, commit_loss


# speedup vs baseline: 1.5335x; 1.0457x over previous
import functools

import jax
import jax.numpy as jnp
from jax import lax
from jax.experimental import pallas as pl
from jax.experimental.pallas import tpu as pltpu
from jax.experimental.pallas import tpu_sc as plsc

D_IN = 768
D_LAT = 64
K = 128

T = 32 * 1024
BT = 2048
NB = T // BT

NB_SC = 1
T_SC = NB_SC * BT

NW = 32
TPW = T_SC // NW
CH = 64
NCH = TPW // CH


def _vq_head(x, encw, encb, cb):
    z = lax.dot_general(x, encw, (((1,), (1,)), ((), ())),
                        preferred_element_type=jnp.float32)
    z = z + encb
    dots = lax.dot_general(z, cb, (((1,), (1,)), ((), ())),
                           preferred_element_type=jnp.float32)
    z2 = jnp.sum(z * z, axis=-1, keepdims=True)
    e2 = jnp.sum(cb * cb, axis=-1)
    dist = z2 - 2.0 * dots + e2[None, :]
    mind = jnp.min(dist, axis=-1, keepdims=True)
    kiota = lax.broadcasted_iota(jnp.int32, dist.shape, 1)
    idx = jnp.min(jnp.where(dist == mind, kiota, K), axis=-1)
    onehot = (kiota == idx[:, None]).astype(jnp.float32)
    q = lax.dot_general(onehot, cb, (((1,), (0,)), ((), ())),
                        preferred_element_type=jnp.float32)
    diff = q - z
    return idx, onehot, jnp.sum(diff * diff)


def _tc_a_body(x_ref, encw_ref, encb_ref, cb_ref, decw_ref, decb_ref,
               idx_ref, loss_ref, deccb_ref):
    i = pl.program_id(0)
    idx, _, part = _vq_head(x_ref[...], encw_ref[...], encb_ref[...],
                            cb_ref[...])
    idx_ref[...] = idx

    @pl.when(i == 0)
    def _():
        loss_ref[...] = jnp.zeros((1, 1), jnp.float32)
        deccb = lax.dot_general(
            cb_ref[...], decw_ref[...], (((1,), (1,)), ((), ())),
            preferred_element_type=jnp.float32)
        deccb_ref[...] = deccb + decb_ref[...]

    loss_ref[...] += part[None, None]


def _tc_a(xf, enc_W, enc_b2, dec_W, dec_b2, codebook):
    return pl.pallas_call(
        _tc_a_body,
        grid=(NB_SC,),
        in_specs=[
            pl.BlockSpec((BT, D_IN), lambda i: (i, 0)),
            pl.BlockSpec((D_LAT, D_IN), lambda i: (0, 0)),
            pl.BlockSpec((1, D_LAT), lambda i: (0, 0)),
            pl.BlockSpec((K, D_LAT), lambda i: (0, 0)),
            pl.BlockSpec((D_IN, D_LAT), lambda i: (0, 0)),
            pl.BlockSpec((1, D_IN), lambda i: (0, 0)),
        ],
        out_specs=[
            pl.BlockSpec((BT,), lambda i: (i,)),
            pl.BlockSpec((1, 1), lambda i: (0, 0)),
            pl.BlockSpec((K, D_IN), lambda i: (0, 0)),
        ],
        out_shape=[
            jax.ShapeDtypeStruct((T_SC,), jnp.int32),
            jax.ShapeDtypeStruct((1, 1), jnp.float32),
            jax.ShapeDtypeStruct((K, D_IN), jnp.float32),
        ],
    )(xf, enc_W, enc_b2, codebook, dec_W, dec_b2)


def _tc_b_body(x_ref, encw_ref, encb_ref, cb_ref, deccb_ref, alias_ref,
               idx_ref, loss_ref, out_ref):
    i = pl.program_id(0)
    idx, onehot, part = _vq_head(x_ref[...], encw_ref[...], encb_ref[...],
                                 cb_ref[...])
    idx_ref[...] = idx
    out_ref[...] = lax.dot_general(
        onehot, deccb_ref[...], (((1,), (0,)), ((), ())),
        preferred_element_type=jnp.float32)

    @pl.when(i == 0)
    def _():
        loss_ref[...] = jnp.zeros((1, 1), jnp.float32)

    loss_ref[...] += part[None, None]


def _tc_b(xf, enc_W, enc_b2, codebook, dec_cb, out_sc):
    return pl.pallas_call(
        _tc_b_body,
        grid=(NB - NB_SC,),
        in_specs=[
            pl.BlockSpec((BT, D_IN), lambda i: (i + NB_SC, 0)),
            pl.BlockSpec((D_LAT, D_IN), lambda i: (0, 0)),
            pl.BlockSpec((1, D_LAT), lambda i: (0, 0)),
            pl.BlockSpec((K, D_LAT), lambda i: (0, 0)),
            pl.BlockSpec((K, D_IN), lambda i: (0, 0)),
            pl.BlockSpec(memory_space=pl.ANY),
        ],
        out_specs=[
            pl.BlockSpec((BT,), lambda i: (i,)),
            pl.BlockSpec((1, 1), lambda i: (0, 0)),
            pl.BlockSpec((BT, D_IN), lambda i: (i + NB_SC, 0)),
        ],
        out_shape=[
            jax.ShapeDtypeStruct((T - T_SC,), jnp.int32),
            jax.ShapeDtypeStruct((1, 1), jnp.float32),
            jax.ShapeDtypeStruct((T, D_IN), jnp.float32),
        ],
        input_output_aliases={5: 2},
    )(xf, enc_W, enc_b2, codebook, dec_cb, out_sc)


def _sc_gather(dec_cb, idx3):
    mesh = plsc.VectorSubcoreMesh(core_axis_name="c", subcore_axis_name="s")

    @functools.partial(
        pl.kernel,
        out_type=jax.ShapeDtypeStruct((T, D_IN), jnp.float32),
        mesh=mesh,
        scratch_types=(
            [pltpu.VMEM((NCH, CH), jnp.int32)]
            + [pltpu.VMEM((CH, D_IN), jnp.float32) for _ in range(2)]
            + [pltpu.SemaphoreType.DMA for _ in range(4)]
        ),
    )
    def body(deccb_hbm, idx_hbm, out_hbm, idx_v, buf0, buf1,
             gsem0, gsem1, psem0, psem1):
        wid = lax.axis_index("s") * 2 + lax.axis_index("c")
        pltpu.sync_copy(idx_hbm.at[wid], idx_v)
        base = wid * TPW
        bufs = (buf0, buf1)
        gsems = (gsem0, gsem1)
        psems = (psem0, psem1)
        gather = [None, None]
        put = [None, None]
        gather[0] = pltpu.async_copy(deccb_hbm.at[idx_v.at[0]], buf0, gsem0)
        for c in range(NCH):
            b = c & 1
            nb = 1 - b
            if c + 1 < NCH:
                if put[nb] is not None:
                    put[nb].wait()
                    put[nb] = None
                gather[nb] = pltpu.async_copy(
                    deccb_hbm.at[idx_v.at[c + 1]], bufs[nb], gsems[nb])
            gather[b].wait()
            put[b] = pltpu.async_copy(
                bufs[b], out_hbm.at[pl.ds(base + c * CH, CH)], psems[b])
        for b in (0, 1):
            if put[b] is not None:
                put[b].wait()

    return body(dec_cb, idx3)


def kernel(x, enc_W, enc_b, dec_W, dec_b, codebook):
    B, N, _ = x.shape
    xf = x.reshape(T, D_IN)
    enc_b2 = enc_b.reshape(1, D_LAT)
    dec_b2 = dec_b.reshape(1, D_IN)
    idx_a, loss_a, dec_cb = _tc_a(xf, enc_W, enc_b2, dec_W, dec_b2, codebook)
    idx3 = idx_a.reshape(NW, NCH, CH)
    out_sc = _sc_gather(dec_cb, idx3)
    idx_b, loss_b, out_flat = _tc_b(xf, enc_W, enc_b2, codebook, dec_cb,
                                    out_sc)
    out = out_flat.reshape(B, N, D_IN)
    indices = jnp.concatenate([idx_a, idx_b]).reshape(B, N)
    commit_loss = (loss_a[0, 0] + loss_b[0, 0]) / jnp.float32(T * D_LAT)
    return out, indices, commit_loss
